# baseline (speedup 1.0000x reference)
import functools

import jax
import jax.numpy as jnp
from jax import lax
from jax.experimental import pallas as pl
from jax.experimental.pallas import tpu as pltpu
from jax.experimental.pallas import tpu_sc as plsc

N = 10000
E = 320000
D = 128
C = 16
K = 10
ALPHA = 0.1

NS = 16
NP = 10240
RPT = NP // NS
CH = 256
NCH = 80
EPT = NCH * CH
PADN = NS * EPT - E
NDUMP = 16
SLOTS = 8
LOOK = 4


def _tc_linear(x, w, b):
    def body(x_ref, w_ref, b_ref, o_ref):
        o_ref[...] = (
            jnp.dot(x_ref[...], w_ref[...], preferred_element_type=jnp.float32)
            + b_ref[...]
        )

    return pl.pallas_call(
        body,
        out_shape=jax.ShapeDtypeStruct((N, C), jnp.float32),
    )(x, w, b.reshape(1, C))


def _tc_log_softmax(r):
    def body(r_ref, o_ref):
        v = r_ref[...]
        m = jnp.max(v, axis=1, keepdims=True)
        e = jnp.exp(v - m)
        s = jnp.sum(e, axis=1, keepdims=True)
        o_ref[...] = (v - m) - jnp.log(s)

    return pl.pallas_call(
        body,
        out_shape=jax.ShapeDtypeStruct((N, C), jnp.float32),
    )(r)


def _rsqrt16(x):
    xi = lax.bitcast_convert_type(x, jnp.int32)
    yi = jnp.int32(0x5F3759DF) - (xi >> 1)
    y = lax.bitcast_convert_type(yi, jnp.float32)
    for _ in range(3):
        y = y * (1.5 - 0.5 * x * y * y)
    return jnp.where(x == 0.0, 1.0, y)


def _sc_prop(rows3, cols3, h, cv):
    mesh = plsc.VectorSubcoreMesh(
        core_axis_name="c", subcore_axis_name="s", num_cores=1
    )

    @functools.partial(
        pl.kernel,
        out_type=(
            jax.ShapeDtypeStruct((NP, C), jnp.float32),
            jax.ShapeDtypeStruct((NP, C), jnp.float32),
        ),
        mesh=mesh,
        compiler_params=pltpu.CompilerParams(use_tc_tiling_on_sc=False),
        scratch_types=[
            pltpu.VMEM_SHARED((NP, C), jnp.float32),
            pltpu.VMEM((NCH, CH), jnp.int32),
            pltpu.VMEM((NCH, CH), jnp.int32),
            pltpu.VMEM((SLOTS, CH, C), jnp.float32),
            pltpu.SemaphoreType.DMA((SLOTS,)),
            pltpu.SemaphoreType.DMA((SLOTS,)),
            pltpu.SemaphoreType.DMA,
            pltpu.VMEM((RPT, C), jnp.float32),
            pltpu.VMEM((RPT, C), jnp.float32),
            pltpu.VMEM((RPT, C), jnp.float32),
            pltpu.VMEM((RPT, C), jnp.float32),
            pltpu.SMEM((RPT,), jnp.float32),
            pltpu.VMEM((K + 1, C), jnp.float32),
        ],
    )
    def body(rows_hbm, cols_hbm, h_hbm, cv_hbm, out_hbm, w_hbm,
             R_sh, rows_v, cols_v, gbuf, gsem, ssem, ssem0, u_prev, u_pp,
             acc, wbuf, dis_v, cvv):
        wid = lax.axis_index("s")
        base = wid * RPT

        pltpu.sync_copy(rows_hbm.at[wid], rows_v)
        pltpu.sync_copy(cols_hbm.at[wid], cols_v)
        pltpu.sync_copy(cv_hbm, cvv)

        ones_row = jnp.full((C,), 1.0, dtype=jnp.float32)
        zero_row = jnp.zeros((C,), dtype=jnp.float32)

        @plsc.parallel_loop(0, CH, unroll=8)
        def fill_ones(r):
            gbuf[0, r, :] = ones_row

        def fill_zero_wbuf():
            @plsc.parallel_loop(0, RPT, unroll=8)
            def _z(r):
                wbuf[r, :] = zero_row

        fill_zero_wbuf()

        pltpu.sync_copy(wbuf, R_sh.at[pl.ds(base, RPT)])
        plsc.subcore_barrier()

        def deg_fire(j, _):
            pltpu.async_copy(gbuf.at[0], R_sh.at[cols_v.at[j]], ssem0,
                             add=True)
            return _

        lax.fori_loop(0, NCH, deg_fire, None)

        def deg_drain(j, _):
            pltpu.make_async_copy(
                gbuf.at[0], R_sh.at[cols_v.at[0]], ssem0
            ).wait()
            return _

        lax.fori_loop(0, NCH, deg_drain, None)
        plsc.subcore_barrier()

        pltpu.sync_copy(R_sh.at[pl.ds(base, RPT)], wbuf)

        @plsc.parallel_loop(0, RPT, unroll=4)
        def mk_dis(r):
            deg = wbuf[r, :]
            dis = _rsqrt16(deg)
            dis_v[r] = dis[0]

        pltpu.sync_copy(h_hbm.at[pl.ds(base, RPT)], wbuf)
        cv0 = cvv[0, :]

        @plsc.parallel_loop(0, RPT, unroll=4)
        def mk_u0(r):
            d = dis_v[r]
            u0 = wbuf[r, :] / d
            u_prev[r, :] = u0
            acc[r, :] = cv0 * u0
            wbuf[r, :] = (d * d) * u0

        pltpu.sync_copy(wbuf, w_hbm.at[pl.ds(base, RPT)])
        fill_zero_wbuf()
        pltpu.sync_copy(wbuf, R_sh.at[pl.ds(base, RPT)])

        def gfire(j, s):
            pltpu.async_copy(w_hbm.at[rows_v.at[j]], gbuf.at[s], gsem.at[s])

        def gwait(s):
            pltpu.make_async_copy(
                w_hbm.at[rows_v.at[0]], gbuf.at[s], gsem.at[s]
            ).wait()

        def sfire(j, s):
            pltpu.async_copy(
                gbuf.at[s], R_sh.at[cols_v.at[j]], ssem.at[s], add=True
            )

        def swait(s):
            pltpu.make_async_copy(
                gbuf.at[s], R_sh.at[cols_v.at[0]], ssem.at[s]
            ).wait()

        up, upp = u_prev, u_pp
        for k in range(1, K + 1):
            plsc.subcore_barrier()

            for b in range(LOOK):
                gfire(b, b)

            for b in range(SLOTS):
                gwait(b)
                sfire(b, b)
                s2 = (b + LOOK) % SLOTS
                if b >= LOOK:
                    swait(s2)
                gfire(b + LOOK, s2)

            def midgroup(g, _):
                for b in range(SLOTS):
                    jj = g * SLOTS + b
                    gwait(b)
                    sfire(jj, b)
                    s2 = (b + LOOK) % SLOTS
                    swait(s2)
                    gfire(jj + LOOK, s2)
                return _

            lax.fori_loop(1, NCH // SLOTS - 1, midgroup, None)

            G = NCH - SLOTS
            for b in range(SLOTS):
                j = G + b
                gwait(b)
                sfire(j, b)
                if j + LOOK < NCH:
                    s2 = (b + LOOK) % SLOTS
                    swait(s2)
                    gfire(j + LOOK, s2)
            for b in range(SLOTS):
                swait(b)
            plsc.subcore_barrier()

            pltpu.sync_copy(R_sh.at[pl.ds(base, RPT)], wbuf)
            cvk = cvv[k, :]
            a = 1.0 if k == 1 else 2.0

            @plsc.parallel_loop(0, RPT, unroll=4)
            def upd(r):
                unew = a * wbuf[r, :]
                upp[r, :] = unew
                acc[r, :] = acc[r, :] + cvk * unew
                d = dis_v[r]
                wbuf[r, :] = (d * d) * unew

            if k < K:
                pltpu.sync_copy(wbuf, w_hbm.at[pl.ds(base, RPT)])

                @plsc.parallel_loop(0, RPT, unroll=8)
                def seedr(r):
                    wbuf[r, :] = -0.5 * up[r, :]

                pltpu.sync_copy(wbuf, R_sh.at[pl.ds(base, RPT)])
            up, upp = upp, up

        @plsc.parallel_loop(0, RPT, unroll=8)
        def fin(r):
            wbuf[r, :] = dis_v[r] * acc[r, :]
        pltpu.sync_copy(wbuf, out_hbm.at[pl.ds(base, RPT)])

    return body(rows3, cols3, h, cv)


def kernel(x, edge_index, W, b, lap_coefs, mf_weights):
    h = _tc_linear(x, W, b)
    hp = jnp.pad(h, ((0, NP - N), (0, 0)))

    lc = jnp.cumprod(ALPHA * jnp.tanh(lap_coefs))
    cv = jnp.concatenate(
        [mf_weights[0, :1, :], lc[: K, None] * mf_weights[0, 1:, :]], axis=0
    ).astype(jnp.float32)

    row = edge_index[0]
    col = edge_index[1]
    spread = (jnp.arange(PADN, dtype=jnp.int32) % NDUMP).astype(jnp.int32)
    prow = jnp.concatenate([row, spread])
    pcol = jnp.concatenate([col, N + spread])
    rows3 = prow.reshape(NS, NCH, CH)
    cols3 = pcol.reshape(NS, NCH, CH)

    ret, _w_unused = _sc_prop(rows3, cols3, hp, cv)
    return _tc_log_softmax(ret[:N])


# speedup vs baseline: 1.5305x; 1.5305x over previous
import functools

import jax
import jax.numpy as jnp
from jax import lax
from jax.experimental import pallas as pl
from jax.experimental.pallas import tpu as pltpu
from jax.experimental.pallas import tpu_sc as plsc

N = 10000
E = 320000
D = 128
C = 16
K = 10
ALPHA = 0.1

NS = 16
NP = 10240
RPT = NP // NS
CH = 128
NCH = 160
EPT = NCH * CH
PADN = NS * EPT - E
NDUMP = 16
SLOTS = 8
LOOK = 4


def _tc_linear(x, w, b):
    def body(x_ref, w_ref, b_ref, o_ref):
        o_ref[...] = (
            jnp.dot(x_ref[...], w_ref[...], preferred_element_type=jnp.float32)
            + b_ref[...]
        )

    return pl.pallas_call(
        body,
        out_shape=jax.ShapeDtypeStruct((N, C), jnp.float32),
    )(x, w, b.reshape(1, C))


def _tc_log_softmax(r):
    def body(r_ref, o_ref):
        v = r_ref[...]
        m = jnp.max(v, axis=1, keepdims=True)
        e = jnp.exp(v - m)
        s = jnp.sum(e, axis=1, keepdims=True)
        o_ref[...] = (v - m) - jnp.log(s)

    return pl.pallas_call(
        body,
        out_shape=jax.ShapeDtypeStruct((N, C), jnp.float32),
    )(r)


def _rsqrt16(x):
    xi = lax.bitcast_convert_type(x, jnp.int32)
    yi = jnp.int32(0x5F3759DF) - (xi >> 1)
    y = lax.bitcast_convert_type(yi, jnp.float32)
    for _ in range(3):
        y = y * (1.5 - 0.5 * x * y * y)
    return jnp.where(x == 0.0, 1.0, y)


def _sc_prop(rows3, cols3, h, cv):
    mesh = plsc.VectorSubcoreMesh(
        core_axis_name="c", subcore_axis_name="s", num_cores=1
    )

    @functools.partial(
        pl.kernel,
        out_type=jax.ShapeDtypeStruct((NP, C), jnp.float32),
        mesh=mesh,
        compiler_params=pltpu.CompilerParams(use_tc_tiling_on_sc=False),
        scratch_types=[
            pltpu.VMEM_SHARED((NP, C), jnp.float32),
            pltpu.VMEM_SHARED((NP, C), jnp.float32),
            pltpu.VMEM((NCH, CH), jnp.int32),
            pltpu.VMEM((NCH, CH), jnp.int32),
            pltpu.VMEM((SLOTS, CH, C), jnp.float32),
            pltpu.SemaphoreType.DMA((SLOTS,)),
            pltpu.SemaphoreType.DMA((SLOTS,)),
            pltpu.SemaphoreType.DMA,
            pltpu.VMEM((RPT, C), jnp.float32),
            pltpu.VMEM((RPT, C), jnp.float32),
            pltpu.VMEM((RPT, C), jnp.float32),
            pltpu.VMEM((RPT, C), jnp.float32),
            pltpu.SMEM((RPT,), jnp.float32),
            pltpu.VMEM((K + 1, C), jnp.float32),
        ],
    )
    def body(rows_hbm, cols_hbm, h_hbm, cv_hbm, out_hbm,
             W_sh, R_sh, rows_v, cols_v, gbuf, gsem, ssem, ssem0, u_prev,
             u_pp, acc, wbuf, dis_v, cvv):
        wid = lax.axis_index("s")
        base = wid * RPT

        pltpu.sync_copy(rows_hbm.at[wid], rows_v)
        pltpu.sync_copy(cols_hbm.at[wid], cols_v)
        pltpu.sync_copy(cv_hbm, cvv)

        ones_row = jnp.full((C,), 1.0, dtype=jnp.float32)
        zero_row = jnp.zeros((C,), dtype=jnp.float32)

        @plsc.parallel_loop(0, CH, unroll=8)
        def fill_ones(r):
            gbuf[0, r, :] = ones_row

        def fill_zero_wbuf():
            @plsc.parallel_loop(0, RPT, unroll=8)
            def _z(r):
                wbuf[r, :] = zero_row

        fill_zero_wbuf()

        pltpu.sync_copy(wbuf, R_sh.at[pl.ds(base, RPT)])
        plsc.subcore_barrier()

        def deg_fire(j, _):
            pltpu.async_copy(gbuf.at[0], R_sh.at[cols_v.at[j]], ssem0,
                             add=True)
            return _

        lax.fori_loop(0, NCH, deg_fire, None)

        def deg_drain(j, _):
            pltpu.make_async_copy(
                gbuf.at[0], R_sh.at[cols_v.at[0]], ssem0
            ).wait()
            return _

        lax.fori_loop(0, NCH, deg_drain, None)
        plsc.subcore_barrier()

        pltpu.sync_copy(R_sh.at[pl.ds(base, RPT)], wbuf)

        @plsc.parallel_loop(0, RPT, unroll=4)
        def mk_dis(r):
            deg = wbuf[r, :]
            dis = _rsqrt16(deg)
            dis_v[r] = dis[0]

        pltpu.sync_copy(h_hbm.at[pl.ds(base, RPT)], wbuf)
        cv0 = cvv[0, :]

        @plsc.parallel_loop(0, RPT, unroll=4)
        def mk_u0(r):
            d = dis_v[r]
            u0 = wbuf[r, :] / d
            u_prev[r, :] = u0
            acc[r, :] = cv0 * u0
            wbuf[r, :] = (d * d) * u0

        pltpu.sync_copy(wbuf, W_sh.at[pl.ds(base, RPT)])
        fill_zero_wbuf()
        pltpu.sync_copy(wbuf, R_sh.at[pl.ds(base, RPT)])

        def gfire(j, s):
            pltpu.async_copy(W_sh.at[rows_v.at[j]], gbuf.at[s], gsem.at[s])

        def gwait(s):
            pltpu.make_async_copy(
                W_sh.at[rows_v.at[0]], gbuf.at[s], gsem.at[s]
            ).wait()

        def sfire(j, s):
            pltpu.async_copy(
                gbuf.at[s], R_sh.at[cols_v.at[j]], ssem.at[s], add=True
            )

        def swait(s):
            pltpu.make_async_copy(
                gbuf.at[s], R_sh.at[cols_v.at[0]], ssem.at[s]
            ).wait()

        up, upp = u_prev, u_pp
        for k in range(1, K + 1):
            plsc.subcore_barrier()

            for b in range(LOOK):
                gfire(b, b)

            for b in range(SLOTS):
                gwait(b)
                sfire(b, b)
                s2 = (b + LOOK) % SLOTS
                if b >= LOOK:
                    swait(s2)
                gfire(b + LOOK, s2)

            def midgroup(g, _):
                for b in range(SLOTS):
                    jj = g * SLOTS + b
                    gwait(b)
                    sfire(jj, b)
                    s2 = (b + LOOK) % SLOTS
                    swait(s2)
                    gfire(jj + LOOK, s2)
                return _

            lax.fori_loop(1, NCH // SLOTS - 1, midgroup, None)

            G = NCH - SLOTS
            for b in range(SLOTS):
                j = G + b
                gwait(b)
                sfire(j, b)
                if j + LOOK < NCH:
                    s2 = (b + LOOK) % SLOTS
                    swait(s2)
                    gfire(j + LOOK, s2)
            for b in range(SLOTS):
                swait(b)
            plsc.subcore_barrier()

            pltpu.sync_copy(R_sh.at[pl.ds(base, RPT)], wbuf)
            cvk = cvv[k, :]
            a = 1.0 if k == 1 else 2.0

            @plsc.parallel_loop(0, RPT, unroll=4)
            def upd(r):
                unew = a * wbuf[r, :]
                upp[r, :] = unew
                acc[r, :] = acc[r, :] + cvk * unew
                d = dis_v[r]
                wbuf[r, :] = (d * d) * unew

            if k < K:
                pltpu.sync_copy(wbuf, W_sh.at[pl.ds(base, RPT)])

                @plsc.parallel_loop(0, RPT, unroll=8)
                def seedr(r):
                    wbuf[r, :] = -0.5 * up[r, :]

                pltpu.sync_copy(wbuf, R_sh.at[pl.ds(base, RPT)])
            up, upp = upp, up

        @plsc.parallel_loop(0, RPT, unroll=8)
        def fin(r):
            wbuf[r, :] = dis_v[r] * acc[r, :]
        pltpu.sync_copy(wbuf, out_hbm.at[pl.ds(base, RPT)])

    return body(rows3, cols3, h, cv)


def kernel(x, edge_index, W, b, lap_coefs, mf_weights):
    h = _tc_linear(x, W, b)
    hp = jnp.pad(h, ((0, NP - N), (0, 0)))

    lc = jnp.cumprod(ALPHA * jnp.tanh(lap_coefs))
    cv = jnp.concatenate(
        [mf_weights[0, :1, :], lc[: K, None] * mf_weights[0, 1:, :]], axis=0
    ).astype(jnp.float32)

    row = edge_index[0]
    col = edge_index[1]
    spread = (jnp.arange(PADN, dtype=jnp.int32) % NDUMP).astype(jnp.int32)
    prow = jnp.concatenate([row, spread])
    pcol = jnp.concatenate([col, N + spread])
    rows3 = prow.reshape(NS, NCH, CH)
    cols3 = pcol.reshape(NS, NCH, CH)

    ret = _sc_prop(rows3, cols3, hp, cv)
    return _tc_log_softmax(ret[:N])


# speedup vs baseline: 1.5496x; 1.0125x over previous
import functools

import jax
import jax.numpy as jnp
from jax import lax
from jax.experimental import pallas as pl
from jax.experimental.pallas import tpu as pltpu
from jax.experimental.pallas import tpu_sc as plsc

N = 10000
E = 320000
D = 128
C = 16
K = 10
ALPHA = 0.1

NS = 16
NP = 10240
RPT = NP // NS
CH = 125
NCH = 160
EPT = NCH * CH
SLOTS = 8
LOOK = 4


def _tc_linear(x, w, b):
    def body(x_ref, w_ref, b_ref, o_ref):
        o_ref[...] = (
            jnp.dot(x_ref[...], w_ref[...], preferred_element_type=jnp.float32)
            + b_ref[...]
        )

    return pl.pallas_call(
        body,
        out_shape=jax.ShapeDtypeStruct((N, C), jnp.float32),
    )(x, w, b.reshape(1, C))


def _tc_log_softmax(r):
    def body(r_ref, o_ref):
        v = r_ref[...]
        m = jnp.max(v, axis=1, keepdims=True)
        e = jnp.exp(v - m)
        s = jnp.sum(e, axis=1, keepdims=True)
        o_ref[...] = (v - m) - jnp.log(s)

    return pl.pallas_call(
        body,
        out_shape=jax.ShapeDtypeStruct((N, C), jnp.float32),
    )(r)


def _rsqrt16(x):
    xi = lax.bitcast_convert_type(x, jnp.int32)
    yi = jnp.int32(0x5F3759DF) - (xi >> 1)
    y = lax.bitcast_convert_type(yi, jnp.float32)
    for _ in range(3):
        y = y * (1.5 - 0.5 * x * y * y)
    return jnp.where(x == 0.0, 1.0, y)


def _sc_prop(rows3, cols3, h, cv):
    mesh = plsc.VectorSubcoreMesh(
        core_axis_name="c", subcore_axis_name="s", num_cores=1
    )

    @functools.partial(
        pl.kernel,
        out_type=jax.ShapeDtypeStruct((NP, C), jnp.float32),
        mesh=mesh,
        compiler_params=pltpu.CompilerParams(use_tc_tiling_on_sc=False),
        scratch_types=[
            pltpu.VMEM_SHARED((NP, C), jnp.float32),
            pltpu.VMEM_SHARED((NP, C), jnp.float32),
            pltpu.VMEM((NCH, CH), jnp.int32),
            pltpu.VMEM((NCH, CH), jnp.int32),
            pltpu.VMEM((SLOTS, CH, C), jnp.float32),
            pltpu.SemaphoreType.DMA((SLOTS,)),
            pltpu.SemaphoreType.DMA((SLOTS,)),
            pltpu.SemaphoreType.DMA,
            pltpu.VMEM((RPT, C), jnp.float32),
            pltpu.VMEM((RPT, C), jnp.float32),
            pltpu.VMEM((RPT, C), jnp.float32),
            pltpu.VMEM((RPT, C), jnp.float32),
            pltpu.SMEM((RPT,), jnp.float32),
            pltpu.VMEM((K + 1, C), jnp.float32),
        ],
    )
    def body(rows_hbm, cols_hbm, h_hbm, cv_hbm, out_hbm,
             W_sh, R_sh, rows_v, cols_v, gbuf, gsem, ssem, ssem0, u_prev,
             u_pp, acc, wbuf, dis_v, cvv):
        wid = lax.axis_index("s")
        base = wid * RPT

        pltpu.sync_copy(rows_hbm.at[wid], rows_v)
        pltpu.sync_copy(cols_hbm.at[wid], cols_v)
        pltpu.sync_copy(cv_hbm, cvv)

        ones_row = jnp.full((C,), 1.0, dtype=jnp.float32)
        zero_row = jnp.zeros((C,), dtype=jnp.float32)

        @plsc.parallel_loop(0, CH, unroll=8)
        def fill_ones(r):
            gbuf[0, r, :] = ones_row

        def fill_zero_wbuf():
            @plsc.parallel_loop(0, RPT, unroll=8)
            def _z(r):
                wbuf[r, :] = zero_row

        fill_zero_wbuf()

        pltpu.sync_copy(wbuf, R_sh.at[pl.ds(base, RPT)])
        plsc.subcore_barrier()

        def deg_fire(j, _):
            pltpu.async_copy(gbuf.at[0], R_sh.at[cols_v.at[j]], ssem0,
                             add=True)
            return _

        lax.fori_loop(0, NCH, deg_fire, None)

        def deg_drain(j, _):
            pltpu.make_async_copy(
                gbuf.at[0], R_sh.at[cols_v.at[0]], ssem0
            ).wait()
            return _

        lax.fori_loop(0, NCH, deg_drain, None)
        plsc.subcore_barrier()

        pltpu.sync_copy(R_sh.at[pl.ds(base, RPT)], wbuf)

        @plsc.parallel_loop(0, RPT, unroll=4)
        def mk_dis(r):
            deg = wbuf[r, :]
            dis = _rsqrt16(deg)
            dis_v[r] = dis[0]

        pltpu.sync_copy(h_hbm.at[pl.ds(base, RPT)], wbuf)
        cv0 = cvv[0, :]

        @plsc.parallel_loop(0, RPT, unroll=4)
        def mk_u0(r):
            d = dis_v[r]
            u0 = wbuf[r, :] / d
            u_prev[r, :] = u0
            acc[r, :] = cv0 * u0
            wbuf[r, :] = (d * d) * u0

        pltpu.sync_copy(wbuf, W_sh.at[pl.ds(base, RPT)])
        fill_zero_wbuf()
        pltpu.sync_copy(wbuf, R_sh.at[pl.ds(base, RPT)])

        def gfire(j, s):
            pltpu.async_copy(W_sh.at[rows_v.at[j]], gbuf.at[s], gsem.at[s])

        def gwait(s):
            pltpu.make_async_copy(
                W_sh.at[rows_v.at[0]], gbuf.at[s], gsem.at[s]
            ).wait()

        def sfire(j, s):
            pltpu.async_copy(
                gbuf.at[s], R_sh.at[cols_v.at[j]], ssem.at[s], add=True
            )

        def swait(s):
            pltpu.make_async_copy(
                gbuf.at[s], R_sh.at[cols_v.at[0]], ssem.at[s]
            ).wait()

        up, upp = u_prev, u_pp
        for k in range(1, K + 1):
            plsc.subcore_barrier()

            for b in range(LOOK):
                gfire(b, b)

            for b in range(SLOTS):
                gwait(b)
                sfire(b, b)
                s2 = (b + LOOK) % SLOTS
                if b >= LOOK:
                    swait(s2)
                gfire(b + LOOK, s2)

            def midgroup(g, _):
                for b in range(SLOTS):
                    jj = g * SLOTS + b
                    gwait(b)
                    sfire(jj, b)
                    s2 = (b + LOOK) % SLOTS
                    swait(s2)
                    gfire(jj + LOOK, s2)
                return _

            lax.fori_loop(1, NCH // SLOTS - 1, midgroup, None)

            G = NCH - SLOTS
            for b in range(SLOTS):
                j = G + b
                gwait(b)
                sfire(j, b)
                if j + LOOK < NCH:
                    s2 = (b + LOOK) % SLOTS
                    swait(s2)
                    gfire(j + LOOK, s2)
            for b in range(SLOTS):
                swait(b)
            plsc.subcore_barrier()

            pltpu.sync_copy(R_sh.at[pl.ds(base, RPT)], wbuf)
            cvk = cvv[k, :]
            a = 1.0 if k == 1 else 2.0

            @plsc.parallel_loop(0, RPT, unroll=4)
            def upd(r):
                unew = a * wbuf[r, :]
                upp[r, :] = unew
                acc[r, :] = acc[r, :] + cvk * unew
                d = dis_v[r]
                wbuf[r, :] = (d * d) * unew

            if k < K:
                pltpu.sync_copy(wbuf, W_sh.at[pl.ds(base, RPT)])

                @plsc.parallel_loop(0, RPT, unroll=8)
                def seedr(r):
                    wbuf[r, :] = -0.5 * up[r, :]

                pltpu.sync_copy(wbuf, R_sh.at[pl.ds(base, RPT)])
            up, upp = upp, up

        @plsc.parallel_loop(0, RPT, unroll=8)
        def fin(r):
            wbuf[r, :] = dis_v[r] * acc[r, :]
        pltpu.sync_copy(wbuf, out_hbm.at[pl.ds(base, RPT)])

    return body(rows3, cols3, h, cv)


def kernel(x, edge_index, W, b, lap_coefs, mf_weights):
    h = _tc_linear(x, W, b)
    hp = jnp.pad(h, ((0, NP - N), (0, 0)))

    lc = jnp.cumprod(ALPHA * jnp.tanh(lap_coefs))
    cv = jnp.concatenate(
        [mf_weights[0, :1, :], lc[: K, None] * mf_weights[0, 1:, :]], axis=0
    ).astype(jnp.float32)

    rows3 = edge_index[0].reshape(NS, NCH, CH)
    cols3 = edge_index[1].reshape(NS, NCH, CH)

    ret = _sc_prop(rows3, cols3, hp, cv)
    return _tc_log_softmax(ret[:N])
